# Initial kernel scaffold; baseline (speedup 1.0000x reference)
import jax
import jax.numpy as jnp
from jax.experimental import pallas as pl


def kernel(scale, pr):
    raise NotImplementedError("write your pallas kernel here")


# speedup vs baseline: 11.6475x; 11.6475x over previous
import jax
import jax.numpy as jnp
import numpy as np
from jax.experimental import pallas as pl
from jax.experimental.pallas import tpu as pltpu

_INT_MIN = np.int32(-2147483648)
_INT_MAX = np.int32(2147483647)


def _encode(x):
    i = jax.lax.bitcast_convert_type(x, jnp.int32)
    return jnp.where(i < 0, i ^ _INT_MAX, i)


def _decode_f32(s):
    m = s ^ _INT_MIN
    f_bits = jnp.where(m < 0, m & _INT_MAX, ~m)
    return jax.lax.bitcast_convert_type(f_bits, jnp.float32)


def _mask_kernel(x_ref, pr_ref, out_ref, k_ref):
    x = x_ref[...]
    n = x.shape[1]
    k_ref[...] = _encode(x)

    pr_s = pr_ref[0, 0]
    pr_eff = jnp.where(pr_s > 10, 10, pr_s).astype(jnp.float32) * 0.1
    pr_bis = 1.0 - pr_eff
    qidx = pr_bis * jnp.float32(n - 1)
    lo_f = jnp.floor(qidx)
    frac = qidx - lo_f
    r = lo_f.astype(jnp.int32)

    def body(j, p):
        bit = 31 - j
        low_mask = (jnp.int32(1) << bit) - 1
        t_u = p | low_mask
        t_s = t_u ^ _INT_MIN
        k = k_ref[...]
        c = jnp.sum((k <= t_s).astype(jnp.int32), axis=1, keepdims=True)
        return jnp.where(c > r, p, p | (jnp.int32(1) << bit))

    p0 = jnp.zeros((x.shape[0], 1), jnp.int32)
    p = jax.lax.fori_loop(0, 32, body, p0)

    v_lo_s = p ^ _INT_MIN
    k = k_ref[...]
    le = k <= v_lo_s
    c_le = jnp.sum(le.astype(jnp.int32), axis=1, keepdims=True)
    gmin = jnp.min(jnp.where(k > v_lo_s, k, _INT_MAX), axis=1, keepdims=True)
    v_hi_s = jnp.where((c_le > r + 1) | (c_le >= n), v_lo_s, gmin)

    x_lo = _decode_f32(v_lo_s)
    x_hi = _decode_f32(v_hi_s)
    q = x_lo + (x_hi - x_lo) * frac

    res = (x >= q).astype(jnp.float32)
    out_ref[...] = jnp.where(pr_s >= 10, 1.0,
                             jnp.where(pr_s == 0, 0.0, res))


def _run(flat, pr_arr, interpret=False):
    b, n = flat.shape
    return pl.pallas_call(
        _mask_kernel,
        out_shape=jax.ShapeDtypeStruct((b, n), jnp.float32),
        scratch_shapes=[pltpu.VMEM((b, n), jnp.int32)],
        interpret=interpret,
    )(flat, pr_arr)


def kernel(scale, pr):
    bs, ch, w, h = scale.shape
    flat = scale.reshape(bs, ch * w * h)
    pr_arr = jnp.asarray(pr, jnp.int32).reshape(1, 1)
    out = _run(flat, pr_arr)
    return out.reshape(bs, ch, w, h)
